# baseline (speedup 1.0000x reference)
import jax
import jax.numpy as jnp
from jax.experimental import pallas as pl
from jax.experimental.pallas import tpu as pltpu

_NLAT_IN = 128
_NLON_IN = 256
_NLAT_OUT = 64
_NLON_OUT = 128
_K = 3
_BAND = 4


def _make_body(BC):
    def _body(st_ref, en_ref, la_ref, r_ref, m0_ref, k_ref, vals_ref,
              x_ref, c_ref, out_ref):
        t = pl.program_id(0)
        is_pole = jnp.logical_or(t == 0, t == _NLAT_OUT - 1)

        @pl.when(is_pole)
        def _pole():
            la0 = jnp.where(t == 0, 0, _NLAT_IN - _BAND)
            for k in range(_K):
                acc = jnp.zeros((_NLON_OUT, BC), jnp.float32)
                for d in range(_BAND):
                    for rr in range(2):
                        acc = acc + jax.lax.dot_general(
                            c_ref[0, k, d, rr],
                            x_ref[la0 + d, rr],
                            (((1,), (0,)), ((), ())),
                            preferred_element_type=jnp.float32,
                            precision=jax.lax.Precision.HIGHEST,
                        )
                out_ref[0, k, :, :] = acc

        @pl.when(jnp.logical_not(is_pole))
        def _interior():
            out_ref[...] = jnp.zeros_like(out_ref)

            def step(e, carry):
                row = x_ref[la_ref[e], r_ref[e], :, :]
                win = pltpu.roll(row, m0_ref[e], axis=0)
                k = k_ref[e]
                out_ref[0, k, :, :] = out_ref[0, k, :, :] + vals_ref[e] * win
                return carry

            jax.lax.fori_loop(st_ref[t], en_ref[t], step, 0)

    return _body


def kernel(x, psi_vals, psi_idx):
    B, C = x.shape[0], x.shape[1]
    BC = B * C

    xrev = x.reshape(BC, _NLAT_IN, _NLON_OUT, 2)[:, :, ::-1, :].transpose(1, 3, 2, 0)

    kk = psi_idx[0].astype(jnp.int32)
    tt = psi_idx[1].astype(jnp.int32)
    cc = psi_idx[2].astype(jnp.int32)
    la = cc // _NLON_IN
    lo = cc - la * _NLON_IN
    r = lo & 1
    m0 = lo >> 1
    la0_t = jnp.clip(2 * tt - 1, 0, _NLAT_IN - _BAND)
    dla = la - la0_t

    pole_sel = jnp.logical_or(tt == 0, tt == _NLAT_OUT - 1)
    pole_id = (tt == _NLAT_OUT - 1).astype(jnp.int32)
    m0r = (_NLON_OUT - m0) & (_NLON_OUT - 1)
    nw = 2 * _K * _BAND * 2 * _NLON_OUT
    widx = ((pole_id * _K + kk) * _BAND + dla) * (2 * _NLON_OUT) + r * _NLON_OUT + m0r
    widx = jnp.where(pole_sel, widx, nw)
    wr = jnp.zeros((nw + 1,), jnp.float32).at[widx].add(psi_vals)[:nw]
    wr = wr.reshape(2, _K, _BAND, 2, _NLON_OUT)
    wr2 = jnp.concatenate([wr, wr], axis=-1)
    cmat = jnp.stack(
        [wr2[..., _NLON_OUT - p:2 * _NLON_OUT - p] for p in range(_NLON_OUT)],
        axis=-2,
    )

    offs = jnp.searchsorted(
        tt, jnp.arange(_NLAT_OUT + 1, dtype=jnp.int32), side='left'
    ).astype(jnp.int32)
    tpole = jnp.logical_or(
        jnp.arange(_NLAT_OUT) == 0, jnp.arange(_NLAT_OUT) == _NLAT_OUT - 1
    )
    st = jnp.where(tpole, 0, offs[:-1])
    en = jnp.where(tpole, 0, offs[1:])

    grid_spec = pltpu.PrefetchScalarGridSpec(
        num_scalar_prefetch=7,
        grid=(_NLAT_OUT,),
        in_specs=[
            pl.BlockSpec(
                (_NLAT_IN, 2, _NLON_OUT, BC),
                lambda t, *_: (0, 0, 0, 0),
            ),
            pl.BlockSpec(
                (1, _K, _BAND, 2, _NLON_OUT, _NLON_OUT),
                lambda t, *_: (jnp.where(t == 0, 0, 1), 0, 0, 0, 0, 0),
            ),
        ],
        out_specs=pl.BlockSpec(
            (1, _K, _NLON_OUT, BC), lambda t, *_: (t, 0, 0, 0)
        ),
    )
    out = pl.pallas_call(
        _make_body(BC),
        grid_spec=grid_spec,
        out_shape=jax.ShapeDtypeStruct((_NLAT_OUT, _K, _NLON_OUT, BC), jnp.float32),
    )(st, en, la, r, m0, kk, psi_vals, xrev, cmat)

    return out.transpose(3, 1, 0, 2).reshape(B, C, _K, _NLAT_OUT, _NLON_OUT)


# speedup vs baseline: 1.1882x; 1.1882x over previous
import jax
import jax.numpy as jnp
from jax.experimental import pallas as pl
from jax.experimental.pallas import tpu as pltpu

_NLAT_IN = 128
_NLON_IN = 256
_NLAT_OUT = 64
_NLON_OUT = 128
_K = 3


def _body(offs_ref, la_ref, r_ref, shift_ref, k_ref, vals_ref, x_ref, out_ref):
    t = pl.program_id(0)
    out_ref[...] = jnp.zeros_like(out_ref)
    e0 = offs_ref[t]
    e1 = offs_ref[t + 1]

    def step(e, carry):
        row = x_ref[la_ref[e], r_ref[e], :, :]
        win = pltpu.roll(row, shift_ref[e], axis=0)
        k = k_ref[e]
        out_ref[0, k, :, :] = out_ref[0, k, :, :] + vals_ref[e] * win
        return carry

    jax.lax.fori_loop(e0, e1, step, 0)


def kernel(x, psi_vals, psi_idx):
    B, C = x.shape[0], x.shape[1]
    BC = B * C

    xpar = x.reshape(BC, _NLAT_IN, _NLON_OUT, 2).transpose(1, 3, 2, 0)

    kk = psi_idx[0].astype(jnp.int32)
    tt = psi_idx[1].astype(jnp.int32)
    cc = psi_idx[2].astype(jnp.int32)
    la = cc // _NLON_IN
    lo = cc - la * _NLON_IN
    r = lo & 1
    m0 = lo >> 1
    shift = (_NLON_OUT - m0) & (_NLON_OUT - 1)
    offs = jnp.searchsorted(
        tt, jnp.arange(_NLAT_OUT + 1, dtype=jnp.int32), side='left'
    ).astype(jnp.int32)

    grid_spec = pltpu.PrefetchScalarGridSpec(
        num_scalar_prefetch=6,
        grid=(_NLAT_OUT,),
        in_specs=[
            pl.BlockSpec(
                (_NLAT_IN, 2, _NLON_OUT, BC),
                lambda t, *_: (0, 0, 0, 0),
            )
        ],
        out_specs=pl.BlockSpec(
            (1, _K, _NLON_OUT, BC), lambda t, *_: (t, 0, 0, 0)
        ),
    )
    out = pl.pallas_call(
        _body,
        grid_spec=grid_spec,
        out_shape=jax.ShapeDtypeStruct((_NLAT_OUT, _K, _NLON_OUT, BC), jnp.float32),
    )(offs, la, r, shift, kk, psi_vals, xpar)

    return out[:, :, ::-1, :].transpose(3, 1, 0, 2).reshape(
        B, C, _K, _NLAT_OUT, _NLON_OUT
    )
